# baseline (speedup 1.0000x reference)
import functools

import jax
import jax.numpy as jnp
from jax import lax
from jax.experimental import pallas as pl
from jax.experimental.pallas import tpu as pltpu
from jax.experimental.pallas import tpu_sc as plsc

_NW = 32
_CHUNK = 256


def _tc_kernel(x, x_embed, y_embed):
    B, H, W, D = x.shape
    BB = 16
    grid = (B // BB,)

    def body(x_ref, xe_ref, ye_ref, o_ref):
        ye = ye_ref[...]
        xe = xe_ref[...]
        pos = ye[:, None, :] + xe[None, :, :]
        o_ref[...] = x_ref[...] + pos[None, :, :, :]

    return pl.pallas_call(
        body,
        grid=grid,
        in_specs=[
            pl.BlockSpec((BB, H, W, D), lambda i: (i, 0, 0, 0)),
            pl.BlockSpec((W, D), lambda i: (0, 0)),
            pl.BlockSpec((H, D), lambda i: (0, 0)),
        ],
        out_specs=pl.BlockSpec((BB, H, W, D), lambda i: (i, 0, 0, 0)),
        out_shape=jax.ShapeDtypeStruct((B, H, W, D), x.dtype),
    )(x, x_embed, y_embed)


def _sc_add(xs, x_embed, y_embed, row0):
    R, D = xs.shape
    rows_per_w = R // _NW
    n_chunks = rows_per_w // _CHUNK
    mesh = plsc.VectorSubcoreMesh(core_axis_name="c", subcore_axis_name="s")

    assert n_chunks % 2 == 0 and rows_per_w % _CHUNK == 0

    @functools.partial(
        pl.kernel,
        out_type=jax.ShapeDtypeStruct((R, D), jnp.float32),
        mesh=mesh,
        scratch_types=[
            pltpu.VMEM((_CHUNK, 128), jnp.float32),
            pltpu.VMEM((_CHUNK, 128), jnp.float32),
            pltpu.VMEM((32, 128), jnp.float32),
            pltpu.VMEM((32, 128), jnp.float32),
            pltpu.SemaphoreType.DMA,
            pltpu.SemaphoreType.DMA,
            pltpu.SemaphoreType.DMA,
            pltpu.SemaphoreType.DMA,
        ],
    )
    def k(xs_hbm, xe_hbm, ye_hbm, out_hbm, buf0, buf1, xe_v, ye_v,
          isem0, isem1, osem0, osem1):
        cid = lax.axis_index("c")
        sid = lax.axis_index("s")
        wid = cid * 16 + sid
        base = wid * rows_per_w
        bufs = (buf0, buf1)
        isems = (isem0, isem1)
        osems = (osem0, osem1)
        pltpu.sync_copy(xe_hbm, xe_v)
        pltpu.sync_copy(ye_hbm, ye_v)

        def start_in(c, b):
            pltpu.async_copy(
                xs_hbm.at[pl.ds(base + c * _CHUNK, _CHUNK)], bufs[b], isems[b])

        def wait_in(c, b):
            pltpu.make_async_copy(
                xs_hbm.at[pl.ds(base + c * _CHUNK, _CHUNK)], bufs[b],
                isems[b]).wait()

        def start_out(c, b):
            pltpu.async_copy(
                bufs[b], out_hbm.at[pl.ds(base + c * _CHUNK, _CHUNK)],
                osems[b])

        def wait_out(c, b):
            pltpu.make_async_copy(
                bufs[b], out_hbm.at[pl.ds(base + c * _CHUNK, _CHUNK)],
                osems[b]).wait()

        def compute(c, buf):
            p = lax.rem(row0 + base + c * _CHUNK, 1024)
            h0 = lax.div(p, 32)
            for hh in range(8):
                h = h0 + hh
                yh = [ye_v[h, pl.ds(db * 16, 16)] for db in range(8)]

                def w_body(w, carry, hh=hh, yh=yh):
                    row = hh * 32 + w
                    for db in range(8):
                        sl = pl.ds(db * 16, 16)
                        buf[row, sl] = buf[row, sl] + yh[db] + xe_v[w, sl]
                    return carry

                lax.fori_loop(0, 32, w_body, 0)

        start_in(0, 0)

        def pair_body(g, carry):
            for b in (0, 1):
                c = 2 * g + b

                @pl.when(c + 1 < n_chunks)
                def _prefetch(c=c, b=b):
                    @pl.when(c >= 1)
                    def _drain(c=c, b=b):
                        wait_out(c - 1, 1 - b)

                    start_in(c + 1, 1 - b)

                wait_in(c, b)
                compute(c, bufs[b])
                start_out(c, b)
            return carry

        lax.fori_loop(0, n_chunks // 2, pair_body, 0)
        wait_out(n_chunks - 2, 0)
        wait_out(n_chunks - 1, 1)

    return k(xs, x_embed, y_embed)


def kernel(x, x_embed, y_embed):
    B, H, W, D = x.shape
    B_SC = 96
    B_TC = B - B_SC
    xs = x[B_TC:].reshape(B_SC * H * W, D)
    out_sc = _sc_add(xs, x_embed, y_embed, B_TC * H * W)
    out_tc = _tc_kernel(x[:B_TC], x_embed, y_embed)
    return jnp.concatenate([out_tc, out_sc.reshape(B_SC, H, W, D)], axis=0)


# speedup vs baseline: 3.2495x; 3.2495x over previous
import functools

import jax
import jax.numpy as jnp
from jax import lax
from jax.experimental import pallas as pl
from jax.experimental.pallas import tpu as pltpu
from jax.experimental.pallas import tpu_sc as plsc

_NW = 32
_CHUNK = 256


def _tc_kernel(x, x_embed, y_embed):
    B, H, W, D = x.shape
    BB = 16
    grid = (B // BB,)

    def body(x_ref, xe_ref, ye_ref, o_ref):
        ye = ye_ref[...]
        xe = xe_ref[...]
        pos = ye[:, None, :] + xe[None, :, :]
        o_ref[...] = x_ref[...] + pos[None, :, :, :]

    return pl.pallas_call(
        body,
        grid=grid,
        in_specs=[
            pl.BlockSpec((BB, H, W, D), lambda i: (i, 0, 0, 0)),
            pl.BlockSpec((W, D), lambda i: (0, 0)),
            pl.BlockSpec((H, D), lambda i: (0, 0)),
        ],
        out_specs=pl.BlockSpec((BB, H, W, D), lambda i: (i, 0, 0, 0)),
        out_shape=jax.ShapeDtypeStruct((B, H, W, D), x.dtype),
    )(x, x_embed, y_embed)


def _sc_add(xs, x_embed, y_embed, row0):
    R, D = xs.shape
    rows_per_w = R // _NW
    n_chunks = rows_per_w // _CHUNK
    mesh = plsc.VectorSubcoreMesh(core_axis_name="c", subcore_axis_name="s")

    assert n_chunks % 2 == 0 and rows_per_w % _CHUNK == 0

    @functools.partial(
        pl.kernel,
        out_type=jax.ShapeDtypeStruct((R, D), jnp.float32),
        mesh=mesh,
        scratch_types=[
            pltpu.VMEM((_CHUNK, 128), jnp.float32),
            pltpu.VMEM((_CHUNK, 128), jnp.float32),
            pltpu.VMEM((32, 128), jnp.float32),
            pltpu.VMEM((32, 128), jnp.float32),
            pltpu.SemaphoreType.DMA,
            pltpu.SemaphoreType.DMA,
            pltpu.SemaphoreType.DMA,
            pltpu.SemaphoreType.DMA,
        ],
    )
    def k(xs_hbm, xe_hbm, ye_hbm, out_hbm, buf0, buf1, xe_v, ye_v,
          isem0, isem1, osem0, osem1):
        cid = lax.axis_index("c")
        sid = lax.axis_index("s")
        wid = cid * 16 + sid
        base = wid * rows_per_w
        bufs = (buf0, buf1)
        isems = (isem0, isem1)
        osems = (osem0, osem1)
        pltpu.sync_copy(xe_hbm, xe_v)
        pltpu.sync_copy(ye_hbm, ye_v)

        def start_in(c, b):
            pltpu.async_copy(
                xs_hbm.at[pl.ds(base + c * _CHUNK, _CHUNK)], bufs[b], isems[b])

        def wait_in(c, b):
            pltpu.make_async_copy(
                xs_hbm.at[pl.ds(base + c * _CHUNK, _CHUNK)], bufs[b],
                isems[b]).wait()

        def start_out(c, b):
            pltpu.async_copy(
                bufs[b], out_hbm.at[pl.ds(base + c * _CHUNK, _CHUNK)],
                osems[b])

        def wait_out(c, b):
            pltpu.make_async_copy(
                bufs[b], out_hbm.at[pl.ds(base + c * _CHUNK, _CHUNK)],
                osems[b]).wait()

        def compute(c, buf):
            p = lax.rem(row0 + base + c * _CHUNK, 1024)
            h0 = lax.div(p, 32)
            for hh in range(8):
                h = h0 + hh
                yh = [ye_v[h, pl.ds(db * 16, 16)] for db in range(8)]

                def w_body(w, carry, hh=hh, yh=yh):
                    row = hh * 32 + w
                    for db in range(8):
                        sl = pl.ds(db * 16, 16)
                        buf[row, sl] = buf[row, sl] + yh[db] + xe_v[w, sl]
                    return carry

                lax.fori_loop(0, 32, w_body, 0)

        start_in(0, 0)

        def pair_body(g, carry):
            for b in (0, 1):
                c = 2 * g + b

                @pl.when(c + 1 < n_chunks)
                def _prefetch(c=c, b=b):
                    @pl.when(c >= 1)
                    def _drain(c=c, b=b):
                        wait_out(c - 1, 1 - b)

                    start_in(c + 1, 1 - b)

                wait_in(c, b)
                compute(c, bufs[b])
                start_out(c, b)
            return carry

        lax.fori_loop(0, n_chunks // 2, pair_body, 0)
        wait_out(n_chunks - 2, 0)
        wait_out(n_chunks - 1, 1)

    return k(xs, x_embed, y_embed)


def kernel(x, x_embed, y_embed):
    return _tc_kernel(x, x_embed, y_embed)
